# baseline (speedup 1.0000x reference)
import functools

import jax
import jax.numpy as jnp
from jax import lax
from jax.experimental import pallas as pl
from jax.experimental.pallas import tpu as pltpu
from jax.experimental.pallas import tpu_sc as plsc

T, B, V, L = 32, 32, 50000, 400
TB = T * B
LANES = 16
NC = 2
CHUNK = 128
NCHUNK = T * L // CHUNK
ROWS_PER_BLK = 1000


@functools.partial(
    pl.pallas_call,
    grid=(V // ROWS_PER_BLK,),
    in_specs=[
        pl.BlockSpec((1, TB), lambda i: (0, 0)),
        pl.BlockSpec((ROWS_PER_BLK, TB), lambda i: (i, 0)),
    ],
    out_specs=pl.BlockSpec((ROWS_PER_BLK, TB), lambda i: (i, 0)),
    out_shape=jax.ShapeDtypeStruct((V, TB), jnp.float32),
)
def _scale(p_ref, v_ref, o_ref):
    o_ref[...] = v_ref[...] * p_ref[...]


def _prep_body(vflat_hbm, attns_hbm, pg_hbm, src_hbm, idx_out, val_out,
               src_v, attn_v, pg_v, tag_v, winl_v, idx_v, val_v, voc_v, sem):
    wid = lax.axis_index("s") * NC + lax.axis_index("c")

    pltpu.sync_copy(src_hbm.at[wid], src_v)
    pltpu.sync_copy(pg_hbm, pg_v)
    pltpu.sync_copy(attns_hbm.at[wid], attn_v)

    for q in range(L // LANES):
        sl = pl.ds(q * LANES, LANES)
        lidx = jax.lax.iota(jnp.int32, LANES) + q * LANES
        plsc.store_scatter(tag_v, [src_v[sl]], lidx)
    for q in range(L // LANES):
        sl = pl.ds(q * LANES, LANES)
        winl_v[sl] = plsc.load_gather(tag_v, [src_v[sl]])

    def idx_body(t, carry):
        tb = t * B + wid
        tbc = (tb // 128) * 1024 + tb % 128
        for q in range(L // LANES):
            sl = pl.ds(q * LANES, LANES)
            s = src_v[sl]
            idx_v[pl.ds(t * L + q * LANES, LANES)] = (
                (s >> 3) * 8192 + (s & 7) * 128 + tbc)
        return carry

    lax.fori_loop(0, T, idx_body, 0)

    for k in range(NCHUNK):
        pltpu.make_async_copy(
            vflat_hbm.at[idx_v.at[pl.ds(k * CHUNK, CHUNK)]],
            voc_v.at[pl.ds(k * CHUNK, CHUNK)],
            sem,
        ).start()
    for k in range(NCHUNK):
        pltpu.make_async_copy(
            vflat_hbm.at[idx_v.at[pl.ds(k * CHUNK, CHUNK)]],
            voc_v.at[pl.ds(k * CHUNK, CHUNK)],
            sem,
        ).wait()

    def val_body(t, carry):
        tb = t * B + wid
        tbx = jnp.zeros((LANES,), jnp.int32) + tb
        p = plsc.load_gather(pg_v, [tbx])
        one_m_p = 1.0 - p
        for q in range(L // LANES):
            sl = pl.ds(q * LANES, LANES)
            o = t * L + q * LANES
            aw = plsc.load_gather(attn_v, [winl_v[sl] + t * L])
            val_v[pl.ds(o, LANES)] = one_m_p * aw + p * voc_v[pl.ds(o, LANES)]
        return carry

    lax.fori_loop(0, T, val_body, 0)

    pltpu.sync_copy(idx_v, idx_out.at[wid])
    pltpu.sync_copy(val_v, val_out.at[wid])


def _scatter_body(idx_hbm, val_hbm, ot_ref, idx_v, val_v, lsem, ssem):
    wid = lax.axis_index("s") * NC + lax.axis_index("c")
    pltpu.async_copy(idx_hbm.at[wid], idx_v, lsem)
    pltpu.async_copy(val_hbm.at[wid], val_v, lsem).wait()
    pltpu.make_async_copy(idx_hbm.at[wid], idx_v, lsem).wait()
    for k in range(NCHUNK):
        pltpu.make_async_copy(
            val_v.at[k], ot_ref.at[idx_v.at[k]], ssem).start()
    for k in range(NCHUNK):
        pltpu.make_async_copy(
            val_v.at[k], ot_ref.at[idx_v.at[k]], ssem).wait()


def _tiled_flat(x2d):
    return (x2d.reshape(V // 8, 8, TB // 128, 128)
            .transpose(0, 2, 1, 3).reshape(-1))


def _tiled_unflat(xflat):
    return (xflat.reshape(V // 8, TB // 128, 8, 128)
            .transpose(0, 2, 1, 3).reshape(V, TB))


@jax.jit
def _dist_gen(vocab_ds, attns_t, p_flat, src_t):
    vt = vocab_ds.T
    ot = _scale(p_flat.reshape(1, TB), vt)

    mesh = plsc.VectorSubcoreMesh(core_axis_name="c", subcore_axis_name="s")
    sc_params = pltpu.CompilerParams(needs_layout_passes=False)

    prep = functools.partial(
        pl.kernel,
        out_type=(
            jax.ShapeDtypeStruct((B, T * L), jnp.int32),
            jax.ShapeDtypeStruct((B, T * L), jnp.float32),
        ),
        mesh=mesh,
        compiler_params=sc_params,
        scratch_types=[
            pltpu.VMEM((L,), jnp.int32),
            pltpu.VMEM((T * L,), jnp.float32),
            pltpu.VMEM((TB,), jnp.float32),
            pltpu.VMEM((V,), jnp.int32),
            pltpu.VMEM((L,), jnp.int32),
            pltpu.VMEM((T * L,), jnp.int32),
            pltpu.VMEM((T * L,), jnp.float32),
            pltpu.VMEM((T * L,), jnp.float32),
            pltpu.SemaphoreType.DMA,
        ],
    )(_prep_body)
    idx, val = prep(_tiled_flat(vt), attns_t, p_flat, src_t)

    ot_ref = jax.new_ref(_tiled_flat(ot))
    scat = functools.partial(
        pl.kernel,
        out_type=(),
        mesh=mesh,
        compiler_params=sc_params,
        scratch_types=[
            pltpu.VMEM((NCHUNK, CHUNK), jnp.int32),
            pltpu.VMEM((NCHUNK, CHUNK), jnp.float32),
            pltpu.SemaphoreType.DMA,
            pltpu.SemaphoreType.DMA,
        ],
    )(_scatter_body)
    scat(idx.reshape(B, NCHUNK, CHUNK), val.reshape(B, NCHUNK, CHUNK), ot_ref)
    return _tiled_unflat(ot_ref[...]).T


def kernel(vocab_ds, attns, p_gens, sources, decoder_batch_len):
    del decoder_batch_len
    p_flat = p_gens.reshape(TB)
    src_t = sources.T.reshape(B, L)
    attns_t = attns.reshape(T, B, L).transpose(1, 0, 2).reshape(B, T * L)
    return _dist_gen(vocab_ds, attns_t, p_flat, src_t)


# speedup vs baseline: 1.0011x; 1.0011x over previous
import functools

import jax
import jax.numpy as jnp
from jax import lax
from jax.experimental import pallas as pl
from jax.experimental.pallas import tpu as pltpu
from jax.experimental.pallas import tpu_sc as plsc

T, B, V, L = 32, 32, 50000, 400
TB = T * B
LANES = 16
NC = 2
CHUNK = 128
NCHUNK = T * L // CHUNK
ROWS_PER_BLK = 1000


@functools.partial(
    pl.pallas_call,
    grid=(V // ROWS_PER_BLK,),
    in_specs=[
        pl.BlockSpec((1, TB), lambda i: (0, 0)),
        pl.BlockSpec((ROWS_PER_BLK, TB), lambda i: (i, 0)),
    ],
    out_specs=pl.BlockSpec((ROWS_PER_BLK, TB), lambda i: (i, 0)),
    out_shape=jax.ShapeDtypeStruct((V, TB), jnp.float32),
)
def _scale(p_ref, v_ref, o_ref):
    o_ref[...] = v_ref[...] * p_ref[...]


def _prep_body(vflat_hbm, attns_hbm, pg_hbm, src_hbm, idx_out, val_out,
               src_v, attn_v, pg_v, tag_v, winl_v, idx_v, val_v, voc_v, sem):
    wid = lax.axis_index("s") * NC + lax.axis_index("c")

    pltpu.sync_copy(src_hbm.at[wid], src_v)
    pltpu.sync_copy(pg_hbm, pg_v)
    pltpu.sync_copy(attns_hbm.at[wid], attn_v)

    for q in range(L // LANES):
        sl = pl.ds(q * LANES, LANES)
        lidx = jax.lax.iota(jnp.int32, LANES) + q * LANES
        plsc.store_scatter(tag_v, [src_v[sl]], lidx)
    for q in range(L // LANES):
        sl = pl.ds(q * LANES, LANES)
        winl_v[sl] = plsc.load_gather(tag_v, [src_v[sl]])

    def idx_body(t, carry):
        tb = t * B + wid
        tbc = (tb // 128) * 1024 + tb % 128
        for q in range(L // LANES):
            sl = pl.ds(q * LANES, LANES)
            s = src_v[sl]
            idx_v[pl.ds(t * L + q * LANES, LANES)] = (
                (s >> 3) * 8192 + (s & 7) * 128 + tbc)
        return carry

    lax.fori_loop(0, T, idx_body, 0)

    for k in range(NCHUNK):
        pltpu.make_async_copy(
            vflat_hbm.at[idx_v.at[pl.ds(k * CHUNK, CHUNK)]],
            voc_v.at[pl.ds(k * CHUNK, CHUNK)],
            sem,
        ).start()
    for k in range(NCHUNK):
        pltpu.make_async_copy(
            vflat_hbm.at[idx_v.at[pl.ds(k * CHUNK, CHUNK)]],
            voc_v.at[pl.ds(k * CHUNK, CHUNK)],
            sem,
        ).wait()

    def val_body(t, carry):
        tb = t * B + wid
        tbx = jnp.zeros((LANES,), jnp.int32) + tb
        p = plsc.load_gather(pg_v, [tbx])
        one_m_p = 1.0 - p
        for q in range(L // LANES):
            sl = pl.ds(q * LANES, LANES)
            o = t * L + q * LANES
            aw = plsc.load_gather(attn_v, [winl_v[sl] + t * L])
            val_v[pl.ds(o, LANES)] = one_m_p * aw + p * voc_v[pl.ds(o, LANES)]
        return carry

    lax.fori_loop(0, T, val_body, 0)

    pltpu.sync_copy(idx_v, idx_out.at[wid])
    pltpu.sync_copy(val_v, val_out.at[wid])


NSEM = 4


def _scatter_body(idx_hbm, val_hbm, ot_ref, idx_v, val_v, lsem, *ssems):
    wid = lax.axis_index("s") * NC + lax.axis_index("c")
    pltpu.async_copy(idx_hbm.at[wid], idx_v, lsem)
    pltpu.async_copy(val_hbm.at[wid], val_v, lsem).wait()
    pltpu.make_async_copy(idx_hbm.at[wid], idx_v, lsem).wait()
    for k in range(NCHUNK):
        pltpu.make_async_copy(
            val_v.at[k], ot_ref.at[idx_v.at[k]], ssems[k % NSEM]).start()
    for k in range(NCHUNK):
        pltpu.make_async_copy(
            val_v.at[k], ot_ref.at[idx_v.at[k]], ssems[k % NSEM]).wait()


def _tiled_flat(x2d):
    return (x2d.reshape(V // 8, 8, TB // 128, 128)
            .transpose(0, 2, 1, 3).reshape(-1))


def _tiled_unflat(xflat):
    return (xflat.reshape(V // 8, TB // 128, 8, 128)
            .transpose(0, 2, 1, 3).reshape(V, TB))


@jax.jit
def _dist_gen(vocab_ds, attns_t, p_flat, src_t):
    vt = vocab_ds.T
    ot = _scale(p_flat.reshape(1, TB), vt)

    mesh = plsc.VectorSubcoreMesh(core_axis_name="c", subcore_axis_name="s")
    sc_params = pltpu.CompilerParams(needs_layout_passes=False)

    prep = functools.partial(
        pl.kernel,
        out_type=(
            jax.ShapeDtypeStruct((B, T * L), jnp.int32),
            jax.ShapeDtypeStruct((B, T * L), jnp.float32),
        ),
        mesh=mesh,
        compiler_params=sc_params,
        scratch_types=[
            pltpu.VMEM((L,), jnp.int32),
            pltpu.VMEM((T * L,), jnp.float32),
            pltpu.VMEM((TB,), jnp.float32),
            pltpu.VMEM((V,), jnp.int32),
            pltpu.VMEM((L,), jnp.int32),
            pltpu.VMEM((T * L,), jnp.int32),
            pltpu.VMEM((T * L,), jnp.float32),
            pltpu.VMEM((T * L,), jnp.float32),
            pltpu.SemaphoreType.DMA,
        ],
    )(_prep_body)
    idx, val = prep(_tiled_flat(vt), attns_t, p_flat, src_t)

    ot_ref = jax.new_ref(_tiled_flat(ot))
    scat = functools.partial(
        pl.kernel,
        out_type=(),
        mesh=mesh,
        compiler_params=sc_params,
        scratch_types=[
            pltpu.VMEM((NCHUNK, CHUNK), jnp.int32),
            pltpu.VMEM((NCHUNK, CHUNK), jnp.float32),
            pltpu.SemaphoreType.DMA,
        ] + [pltpu.SemaphoreType.DMA] * NSEM,
    )(_scatter_body)
    scat(idx.reshape(B, NCHUNK, CHUNK), val.reshape(B, NCHUNK, CHUNK), ot_ref)
    return _tiled_unflat(ot_ref[...]).T


def kernel(vocab_ds, attns, p_gens, sources, decoder_batch_len):
    del decoder_batch_len
    p_flat = p_gens.reshape(TB)
    src_t = sources.T.reshape(B, L)
    attns_t = attns.reshape(T, B, L).transpose(1, 0, 2).reshape(B, T * L)
    return _dist_gen(vocab_ds, attns_t, p_flat, src_t)
